# baseline (speedup 1.0000x reference)
import functools

import jax
import jax.numpy as jnp
import numpy as np
from jax import lax
from jax.experimental import pallas as pl
from jax.experimental.pallas import tpu as pltpu
from jax.experimental.pallas import tpu_sc as plsc

_NC = 2
_NS = 16
_K = 40


def _interleave_perm(h):
    perm = np.empty(h, np.int32)
    for p in range(h // 32):
        for k in range(16):
            perm[32 * p + 2 * k] = 32 * p + k
            perm[32 * p + 2 * k + 1] = 32 * p + 16 + k
    return perm


def _leaky(x):
    return jnp.where(x >= 0, x, 0.1 * x)


def _node_dense_body(cf_ref, w1_ref, b1_ref, w2_ref, b2_ref, ws_ref, wd_ref,
                     child_ref, a_ref, b_ref):
    x = cf_ref[...]
    net = _leaky(_leaky(
        jnp.dot(x, w1_ref[...], preferred_element_type=jnp.float32) + b1_ref[...]))
    net = _leaky(
        jnp.dot(net, w2_ref[...], preferred_element_type=jnp.float32) + b2_ref[...])
    child_ref[...] = net
    a_ref[...] = jnp.dot(net, ws_ref[...], preferred_element_type=jnp.float32)
    b_ref[...] = jnp.dot(net, wd_ref[...], preferred_element_type=jnp.float32)


def _edge_dense_body(ef_ref, w_ref, b_ref, c_ref):
    x = ef_ref[...]
    c_ref[...] = jnp.dot(x, w_ref[...], preferred_element_type=jnp.float32) + b_ref[...]


def _mid_dense_body(p_ref, ws_ref, wd_ref, child_ref, a_ref, b_ref):
    ch = p_ref[0] + p_ref[1]
    child_ref[...] = ch
    a_ref[...] = jnp.dot(ch, ws_ref[...], preferred_element_type=jnp.float32)
    b_ref[...] = jnp.dot(ch, wd_ref[...], preferred_element_type=jnp.float32)


def _final_dense_body(c0_ref, c1_ref, q_ref, w0_ref, w1_ref, w2_ref, bs_ref,
                      out_ref):
    ch2 = q_ref[0] + q_ref[1]
    acc = jnp.dot(c0_ref[...], w0_ref[...], preferred_element_type=jnp.float32)
    acc = acc + jnp.dot(c1_ref[...], w1_ref[...], preferred_element_type=jnp.float32)
    acc = acc + jnp.dot(ch2, w2_ref[...], preferred_element_type=jnp.float32)
    out_ref[...] = _leaky(acc + bs_ref[...])


@functools.lru_cache(maxsize=None)
def _make_edge_pass(n, e, h):
    ept = e // (_NC * _NS)
    n_chunks = ept // _K
    zfull = 624
    mesh = plsc.VectorSubcoreMesh(core_axis_name="c", subcore_axis_name="s")

    nq = 4

    @functools.partial(
        pl.kernel,
        out_type=jax.ShapeDtypeStruct((_NC, n, h), jnp.float32),
        mesh=mesh,
        scratch_types=[
            pltpu.VMEM((nq, _K), jnp.int32),
            pltpu.VMEM((nq, _K), jnp.int32),
            pltpu.VMEM((2, _K, h), jnp.float32),
            pltpu.VMEM((2, _K, h), jnp.float32),
            pltpu.VMEM((2, _K, h), jnp.float32),
            pltpu.VMEM((2, _K, h), jnp.float32),
            pltpu.VMEM_SHARED((n, h), jnp.float32),
            pltpu.SemaphoreType.DMA,
            pltpu.SemaphoreType.DMA,
            pltpu.SemaphoreType.DMA,
            pltpu.SemaphoreType.DMA,
            pltpu.SemaphoreType.DMA,
            pltpu.SemaphoreType.DMA,
            pltpu.SemaphoreType.DMA,
            pltpu.SemaphoreType.DMA,
            pltpu.SemaphoreType.DMA,
        ],
    )
    def edge_pass(a_hbm, b_hbm, c_hbm, src_hbm, dst_hbm, out_hbm,
                  idxs, idxd, bufa, bufb, bufc, bufm, acc,
                  semi, sa0, sa1, sb0, sb1, sc0, sc1, ss0, ss1):
        cid = lax.axis_index("c")
        sid = lax.axis_index("s")
        tile = cid * _NS + sid
        last = sid == _NS - 1
        row0 = sid * zfull
        ebase = tile * ept
        sem_a = (sa0, sa1)
        sem_b = (sb0, sb1)
        sem_c = (sc0, sc1)
        sem_s = (ss0, ss1)

        zero16 = jnp.zeros((16,), jnp.float32)

        def zrow(r, carry):
            for q in range(h // 16):
                bufm[0, r, pl.ds(q * 16, 16)] = zero16
            return carry

        lax.fori_loop(0, _K, zrow, 0)

        ncp = jnp.where(last, 16, 15)

        def zcp(k, carry):
            pltpu.sync_copy(bufm.at[0], acc.at[pl.ds(row0 + k * _K, _K)])
            return carry

        lax.fori_loop(0, ncp, zcp, 0)

        @pl.when(jnp.logical_not(last))
        def _zero_tail():
            pltpu.sync_copy(bufm.at[0, pl.ds(0, zfull - 15 * _K)],
                            acc.at[pl.ds(row0 + 15 * _K, zfull - 15 * _K)])

        plsc.subcore_barrier()

        def load_idx(j, q):
            di = pltpu.async_copy(
                src_hbm.at[pl.ds(ebase + j * _K, _K)], idxs.at[q], semi)
            dj = pltpu.async_copy(
                dst_hbm.at[pl.ds(ebase + j * _K, _K)], idxd.at[q], semi)
            return di, dj

        def issue_gathers(j, q, s):
            pltpu.async_copy(c_hbm.at[pl.ds(ebase + j * _K, _K)],
                             bufc.at[s], sem_c[s])
            pltpu.async_copy(a_hbm.at[idxs.at[q]], bufa.at[s], sem_a[s])
            pltpu.async_copy(b_hbm.at[idxd.at[q]], bufb.at[s], sem_b[s])

        def wait_gathers(s):
            pltpu.make_async_copy(c_hbm.at[pl.ds(0, _K)],
                                  bufc.at[s], sem_c[s]).wait()
            pltpu.make_async_copy(a_hbm.at[pl.ds(0, _K)],
                                  bufa.at[s], sem_a[s]).wait()
            pltpu.make_async_copy(b_hbm.at[pl.ds(0, _K)],
                                  bufb.at[s], sem_b[s]).wait()

        def wait_idx(q):
            pltpu.make_async_copy(src_hbm.at[pl.ds(0, _K)],
                                  idxs.at[q], semi).wait()
            pltpu.make_async_copy(src_hbm.at[pl.ds(0, _K)],
                                  idxd.at[q], semi).wait()

        def wait_scatter(s):
            pltpu.make_async_copy(c_hbm.at[pl.ds(0, _K)],
                                  bufm.at[s], sem_s[s]).wait()

        def make_chunk(j, q, s, wait_scat, pg, pi):
            sn = 1 - s
            qn = (q + 1) % nq
            if pg:
                wait_idx(qn)
                issue_gathers(j + 1, qn, sn)
            if wait_scat:
                wait_scatter(s)
            if pi:
                load_idx(j + 2, (q + 2) % nq)
            wait_gathers(s)

            @plsc.parallel_loop(0, _K, unroll=4)
            def erow(ei):
                for qq in range(h // 16):
                    sl = pl.ds(qq * 16, 16)
                    v = bufc[s, ei, sl] + bufa[s, ei, sl] + bufb[s, ei, sl]
                    bufm[s, ei, sl] = jnp.maximum(v, 0.0)

            pltpu.async_copy(bufm.at[s], acc.at[idxs.at[q]], sem_s[s],
                             add=True)

        d0, d1 = load_idx(0, 0)
        d0.wait()
        d1.wait()
        load_idx(1, 1)
        issue_gathers(0, 0, 0)
        make_chunk(0, 0, 0, False, True, True)
        make_chunk(1, 1, 1, False, True, True)

        def quad(t, carry):
            j = 2 + 4 * t
            make_chunk(j, 2, 0, True, True, True)
            make_chunk(j + 1, 3, 1, True, True, True)
            make_chunk(j + 2, 0, 0, True, True, True)
            make_chunk(j + 3, 1, 1, True, True, True)
            return carry

        lax.fori_loop(0, (n_chunks - 2) // 4 - 1, quad, 0)
        jt = n_chunks - 4
        make_chunk(jt, 2, 0, True, True, True)
        make_chunk(jt + 1, 3, 1, True, True, True)
        make_chunk(jt + 2, 0, 0, True, True, False)
        make_chunk(jt + 3, 1, 1, True, False, False)
        wait_scatter(0)
        wait_scatter(1)

        plsc.subcore_barrier()

        pltpu.sync_copy(acc.at[pl.ds(row0, zfull)],
                        out_hbm.at[cid, pl.ds(row0, zfull)])

        @pl.when(last)
        def _flush_tail():
            pltpu.sync_copy(acc.at[pl.ds(15 * zfull, n - 15 * zfull)],
                            out_hbm.at[cid, pl.ds(15 * zfull, n - 15 * zfull)])

    return edge_pass


def kernel(child_feats, edge_indices, edge_type_onehot, W1, b1, W2, b2,
           We0, be0, We1, be1, Ws, bs):
    n = child_feats.shape[1]
    e = edge_indices.shape[1]
    h = W1.shape[1]
    nfs = Ws.shape[1]

    cf = child_feats[0]
    ef = edge_type_onehot[0]
    src = edge_indices[0, :, 0]
    dst = edge_indices[0, :, 1]

    f32 = jnp.float32
    node_out = [jax.ShapeDtypeStruct((n, h), f32)] * 3

    child0, a0, b0 = pl.pallas_call(
        _node_dense_body,
        out_shape=node_out,
    )(cf, W1, b1.reshape(1, h), W2, b2.reshape(1, h),
      We0[:h], We0[h:2 * h])

    rb = 8000
    et = ef.shape[1]

    def edge_dense(w, b):
        return pl.pallas_call(
            _edge_dense_body,
            grid=(e // rb,),
            in_specs=[
                pl.BlockSpec((rb, et), lambda i: (i, 0)),
                pl.BlockSpec((et, h), lambda i: (0, 0)),
                pl.BlockSpec((1, h), lambda i: (0, 0)),
            ],
            out_specs=pl.BlockSpec((rb, h), lambda i: (i, 0)),
            out_shape=jax.ShapeDtypeStruct((e, h), f32),
        )(ef, w, b.reshape(1, h))

    c0 = edge_dense(We0[2 * h:], be0)

    edge_pass = _make_edge_pass(n, e, h)
    p0 = edge_pass(a0, b0, c0, src, dst)
    c1 = edge_dense(We1[2 * h:], be1)

    child1, a1, b1_ = pl.pallas_call(
        _mid_dense_body,
        out_shape=node_out,
    )(p0, We1[:h], We1[h:2 * h])

    p1 = edge_pass(a1, b1_, c1, src, dst)

    out = pl.pallas_call(
        _final_dense_body,
        out_shape=jax.ShapeDtypeStruct((n, nfs), f32),
    )(child0, child1, p1, Ws[:h], Ws[h:2 * h], Ws[2 * h:], bs.reshape(1, nfs))

    return out


# speedup vs baseline: 1.0341x; 1.0341x over previous
import functools

import jax
import jax.numpy as jnp
import numpy as np
from jax import lax
from jax.experimental import pallas as pl
from jax.experimental.pallas import tpu as pltpu
from jax.experimental.pallas import tpu_sc as plsc

_NC = 2
_NS = 16
_K = 40


def _interleave_perm(h):
    perm = np.empty(h, np.int32)
    for p in range(h // 32):
        for k in range(16):
            perm[32 * p + 2 * k] = 32 * p + k
            perm[32 * p + 2 * k + 1] = 32 * p + 16 + k
    return perm


def _leaky(x):
    return jnp.where(x >= 0, x, 0.1 * x)


def _node_dense_body(cf_ref, w1_ref, b1_ref, w2_ref, b2_ref, ws_ref, wd_ref,
                     child_ref, a_ref, b_ref):
    x = cf_ref[...]
    net = _leaky(_leaky(
        jnp.dot(x, w1_ref[...], preferred_element_type=jnp.float32) + b1_ref[...]))
    net = _leaky(
        jnp.dot(net, w2_ref[...], preferred_element_type=jnp.float32) + b2_ref[...])
    child_ref[...] = net
    a_ref[...] = jnp.dot(net, ws_ref[...], preferred_element_type=jnp.float32)
    b_ref[...] = jnp.dot(net, wd_ref[...], preferred_element_type=jnp.float32)


def _edge_dense_body(ef_ref, w0lo_ref, w0hi_ref, b0lo_ref, b0hi_ref,
                     w1lo_ref, w1hi_ref, b1lo_ref, b1hi_ref, c0_ref, c1_ref):
    x = ef_ref[...]

    def pack(lo, hi):
        ulo = jax.lax.bitcast_convert_type(lo, jnp.uint32)
        uhi = jax.lax.bitcast_convert_type(hi, jnp.uint32)
        ulo = (ulo + 0x8000) >> 16
        uhi = (uhi + 0x8000) >> 16
        return jax.lax.bitcast_convert_type(ulo | (uhi << 16), jnp.int32)

    def half(w_ref, b_ref):
        return jnp.dot(x, w_ref[...], preferred_element_type=jnp.float32) + b_ref[...]

    c0_ref[...] = pack(half(w0lo_ref, b0lo_ref), half(w0hi_ref, b0hi_ref))
    c1_ref[...] = pack(half(w1lo_ref, b1lo_ref), half(w1hi_ref, b1hi_ref))


def _mid_dense_body(p_ref, ws_ref, wd_ref, child_ref, a_ref, b_ref):
    ch = p_ref[0] + p_ref[1]
    child_ref[...] = ch
    a_ref[...] = jnp.dot(ch, ws_ref[...], preferred_element_type=jnp.float32)
    b_ref[...] = jnp.dot(ch, wd_ref[...], preferred_element_type=jnp.float32)


def _final_dense_body(c0_ref, c1_ref, q_ref, w0_ref, w1_ref, w2_ref, bs_ref,
                      out_ref):
    ch2 = q_ref[0] + q_ref[1]
    acc = jnp.dot(c0_ref[...], w0_ref[...], preferred_element_type=jnp.float32)
    acc = acc + jnp.dot(c1_ref[...], w1_ref[...], preferred_element_type=jnp.float32)
    acc = acc + jnp.dot(ch2, w2_ref[...], preferred_element_type=jnp.float32)
    out_ref[...] = _leaky(acc + bs_ref[...])


@functools.lru_cache(maxsize=None)
def _make_edge_pass(n, e, h):
    ept = e // (_NC * _NS)
    n_chunks = ept // _K
    zfull = 624
    mesh = plsc.VectorSubcoreMesh(core_axis_name="c", subcore_axis_name="s")

    nq = 4

    @functools.partial(
        pl.kernel,
        out_type=jax.ShapeDtypeStruct((_NC, n, h), jnp.float32),
        mesh=mesh,
        scratch_types=[
            pltpu.VMEM((nq, _K), jnp.int32),
            pltpu.VMEM((nq, _K), jnp.int32),
            pltpu.VMEM((2, _K, h), jnp.float32),
            pltpu.VMEM((2, _K, h), jnp.float32),
            pltpu.VMEM((2, _K, h // 2), jnp.int32),
            pltpu.VMEM((2, _K, h), jnp.float32),
            pltpu.VMEM_SHARED((n, h), jnp.float32),
            pltpu.SemaphoreType.DMA,
            pltpu.SemaphoreType.DMA,
            pltpu.SemaphoreType.DMA,
            pltpu.SemaphoreType.DMA,
            pltpu.SemaphoreType.DMA,
            pltpu.SemaphoreType.DMA,
            pltpu.SemaphoreType.DMA,
            pltpu.SemaphoreType.DMA,
            pltpu.SemaphoreType.DMA,
        ],
    )
    def edge_pass(a_hbm, b_hbm, c_hbm, src_hbm, dst_hbm, out_hbm,
                  idxs, idxd, bufa, bufb, bufc, bufm, acc,
                  semi, sa0, sa1, sb0, sb1, sc0, sc1, ss0, ss1):
        cid = lax.axis_index("c")
        sid = lax.axis_index("s")
        tile = cid * _NS + sid
        last = sid == _NS - 1
        row0 = sid * zfull
        ebase = tile * ept
        sem_a = (sa0, sa1)
        sem_b = (sb0, sb1)
        sem_c = (sc0, sc1)
        sem_s = (ss0, ss1)

        zero16 = jnp.zeros((16,), jnp.float32)

        def zrow(r, carry):
            for q in range(h // 16):
                bufm[0, r, pl.ds(q * 16, 16)] = zero16
            return carry

        lax.fori_loop(0, _K, zrow, 0)

        ncp = jnp.where(last, 16, 15)

        def zcp(k, carry):
            pltpu.sync_copy(bufm.at[0], acc.at[pl.ds(row0 + k * _K, _K)])
            return carry

        lax.fori_loop(0, ncp, zcp, 0)

        @pl.when(jnp.logical_not(last))
        def _zero_tail():
            pltpu.sync_copy(bufm.at[0, pl.ds(0, zfull - 15 * _K)],
                            acc.at[pl.ds(row0 + 15 * _K, zfull - 15 * _K)])

        plsc.subcore_barrier()

        def load_idx(j, q):
            di = pltpu.async_copy(
                src_hbm.at[pl.ds(ebase + j * _K, _K)], idxs.at[q], semi)
            dj = pltpu.async_copy(
                dst_hbm.at[pl.ds(ebase + j * _K, _K)], idxd.at[q], semi)
            return di, dj

        def issue_gathers(j, q, s):
            pltpu.async_copy(c_hbm.at[pl.ds(ebase + j * _K, _K)],
                             bufc.at[s], sem_c[s])
            pltpu.async_copy(a_hbm.at[idxs.at[q]], bufa.at[s], sem_a[s])
            pltpu.async_copy(b_hbm.at[idxd.at[q]], bufb.at[s], sem_b[s])

        def wait_gathers(s):
            pltpu.make_async_copy(c_hbm.at[pl.ds(0, _K)],
                                  bufc.at[s], sem_c[s]).wait()
            pltpu.make_async_copy(a_hbm.at[pl.ds(0, _K)],
                                  bufa.at[s], sem_a[s]).wait()
            pltpu.make_async_copy(b_hbm.at[pl.ds(0, _K)],
                                  bufb.at[s], sem_b[s]).wait()

        def wait_idx(q):
            pltpu.make_async_copy(src_hbm.at[pl.ds(0, _K)],
                                  idxs.at[q], semi).wait()
            pltpu.make_async_copy(src_hbm.at[pl.ds(0, _K)],
                                  idxd.at[q], semi).wait()

        def wait_scatter(s):
            pltpu.make_async_copy(c_hbm.at[pl.ds(0, _K)],
                                  bufm.at[s], sem_s[s]).wait()

        def make_chunk(j, q, s, wait_scat, pg, pi):
            sn = 1 - s
            qn = (q + 1) % nq
            if pg:
                wait_idx(qn)
                issue_gathers(j + 1, qn, sn)
            if wait_scat:
                wait_scatter(s)
            if pi:
                load_idx(j + 2, (q + 2) % nq)
            wait_gathers(s)

            himask = jnp.int32(-65536)

            @plsc.parallel_loop(0, _K, unroll=4)
            def erow(ei):
                for p in range(h // 32):
                    w = bufc[s, ei, pl.ds(16 * p, 16)]
                    clo = jax.lax.bitcast_convert_type(w << 16, jnp.float32)
                    chi = jax.lax.bitcast_convert_type(w & himask, jnp.float32)
                    sll = pl.ds(16 * p, 16)
                    slh = pl.ds(h // 2 + 16 * p, 16)
                    vl = clo + bufa[s, ei, sll] + bufb[s, ei, sll]
                    vh = chi + bufa[s, ei, slh] + bufb[s, ei, slh]
                    bufm[s, ei, sll] = jnp.maximum(vl, 0.0)
                    bufm[s, ei, slh] = jnp.maximum(vh, 0.0)

            pltpu.async_copy(bufm.at[s], acc.at[idxs.at[q]], sem_s[s],
                             add=True)

        d0, d1 = load_idx(0, 0)
        d0.wait()
        d1.wait()
        load_idx(1, 1)
        issue_gathers(0, 0, 0)
        make_chunk(0, 0, 0, False, True, True)
        make_chunk(1, 1, 1, False, True, True)

        def quad(t, carry):
            j = 2 + 4 * t
            make_chunk(j, 2, 0, True, True, True)
            make_chunk(j + 1, 3, 1, True, True, True)
            make_chunk(j + 2, 0, 0, True, True, True)
            make_chunk(j + 3, 1, 1, True, True, True)
            return carry

        lax.fori_loop(0, (n_chunks - 2) // 4 - 1, quad, 0)
        jt = n_chunks - 4
        make_chunk(jt, 2, 0, True, True, True)
        make_chunk(jt + 1, 3, 1, True, True, True)
        make_chunk(jt + 2, 0, 0, True, True, False)
        make_chunk(jt + 3, 1, 1, True, False, False)
        wait_scatter(0)
        wait_scatter(1)

        plsc.subcore_barrier()

        pltpu.sync_copy(acc.at[pl.ds(row0, zfull)],
                        out_hbm.at[cid, pl.ds(row0, zfull)])

        @pl.when(last)
        def _flush_tail():
            pltpu.sync_copy(acc.at[pl.ds(15 * zfull, n - 15 * zfull)],
                            out_hbm.at[cid, pl.ds(15 * zfull, n - 15 * zfull)])

    return edge_pass


def kernel(child_feats, edge_indices, edge_type_onehot, W1, b1, W2, b2,
           We0, be0, We1, be1, Ws, bs):
    n = child_feats.shape[1]
    e = edge_indices.shape[1]
    h = W1.shape[1]
    nfs = Ws.shape[1]

    cf = child_feats[0]
    ef = edge_type_onehot[0]
    src = edge_indices[0, :, 0]
    dst = edge_indices[0, :, 1]

    f32 = jnp.float32
    node_out = [jax.ShapeDtypeStruct((n, h), f32)] * 3

    child0, a0, b0 = pl.pallas_call(
        _node_dense_body,
        out_shape=node_out,
    )(cf, W1, b1.reshape(1, h), W2, b2.reshape(1, h),
      We0[:h], We0[h:2 * h])

    rb = 8000
    et = ef.shape[1]
    hh = h // 2
    wspec = pl.BlockSpec((et, hh), lambda i: (0, 0))
    bspec = pl.BlockSpec((1, hh), lambda i: (0, 0))
    c0, c1 = pl.pallas_call(
        _edge_dense_body,
        grid=(e // rb,),
        in_specs=[pl.BlockSpec((rb, et), lambda i: (i, 0))]
        + [wspec, wspec, bspec, bspec] * 2,
        out_specs=[pl.BlockSpec((rb, hh), lambda i: (i, 0))] * 2,
        out_shape=[jax.ShapeDtypeStruct((e, hh), jnp.int32)] * 2,
    )(ef,
      We0[2 * h:, :hh], We0[2 * h:, hh:],
      be0[:hh].reshape(1, hh), be0[hh:].reshape(1, hh),
      We1[2 * h:, :hh], We1[2 * h:, hh:],
      be1[:hh].reshape(1, hh), be1[hh:].reshape(1, hh))

    edge_pass = _make_edge_pass(n, e, h)
    p0 = edge_pass(a0, b0, c0, src, dst)

    child1, a1, b1_ = pl.pallas_call(
        _mid_dense_body,
        out_shape=node_out,
    )(p0, We1[:h], We1[h:2 * h])

    p1 = edge_pass(a1, b1_, c1, src, dst)

    out = pl.pallas_call(
        _final_dense_body,
        out_shape=jax.ShapeDtypeStruct((n, nfs), f32),
    )(child0, child1, p1, Ws[:h], Ws[h:2 * h], Ws[2 * h:], bs.reshape(1, nfs))

    return out


# speedup vs baseline: 1.0351x; 1.0010x over previous
import functools

import jax
import jax.numpy as jnp
import numpy as np
from jax import lax
from jax.experimental import pallas as pl
from jax.experimental.pallas import tpu as pltpu
from jax.experimental.pallas import tpu_sc as plsc

_NC = 2
_NS = 16
_K = 40


def _interleave_perm(h):
    perm = np.empty(h, np.int32)
    for p in range(h // 32):
        for k in range(16):
            perm[32 * p + 2 * k] = 32 * p + k
            perm[32 * p + 2 * k + 1] = 32 * p + 16 + k
    return perm


def _leaky(x):
    return jnp.where(x >= 0, x, 0.1 * x)


def _node_dense_body(cf_ref, w1_ref, b1_ref, w2_ref, b2_ref, ws_ref, wd_ref,
                     child_ref, a_ref, b_ref):
    x = cf_ref[...]
    net = _leaky(_leaky(
        jnp.dot(x, w1_ref[...], preferred_element_type=jnp.float32) + b1_ref[...]))
    net = _leaky(
        jnp.dot(net, w2_ref[...], preferred_element_type=jnp.float32) + b2_ref[...])
    child_ref[...] = net
    a_ref[...] = jnp.dot(net, ws_ref[...], preferred_element_type=jnp.float32)
    b_ref[...] = jnp.dot(net, wd_ref[...], preferred_element_type=jnp.float32)


def _edge_dense_body(ef_ref, w0lo_ref, w0hi_ref, b0lo_ref, b0hi_ref,
                     w1lo_ref, w1hi_ref, b1lo_ref, b1hi_ref, c0_ref, c1_ref):
    x = ef_ref[...]

    def pack(lo, hi):
        ulo = jax.lax.bitcast_convert_type(lo, jnp.uint32)
        uhi = jax.lax.bitcast_convert_type(hi, jnp.uint32)
        ulo = (ulo + 0x8000) >> 16
        uhi = (uhi + 0x8000) >> 16
        return jax.lax.bitcast_convert_type(ulo | (uhi << 16), jnp.int32)

    def half(w_ref, b_ref):
        return jnp.dot(x, w_ref[...], preferred_element_type=jnp.float32) + b_ref[...]

    c0_ref[...] = pack(half(w0lo_ref, b0lo_ref), half(w0hi_ref, b0hi_ref))
    c1_ref[...] = pack(half(w1lo_ref, b1lo_ref), half(w1hi_ref, b1hi_ref))


def _mid_dense_body(p_ref, ws_ref, wd_ref, child_ref, a_ref, b_ref):
    ch = p_ref[0] + p_ref[1]
    child_ref[...] = ch
    a_ref[...] = jnp.dot(ch, ws_ref[...], preferred_element_type=jnp.float32)
    b_ref[...] = jnp.dot(ch, wd_ref[...], preferred_element_type=jnp.float32)


def _final_dense_body(c0_ref, c1_ref, q_ref, w0_ref, w1_ref, w2_ref, bs_ref,
                      out_ref):
    ch2 = q_ref[0] + q_ref[1]
    acc = jnp.dot(c0_ref[...], w0_ref[...], preferred_element_type=jnp.float32)
    acc = acc + jnp.dot(c1_ref[...], w1_ref[...], preferred_element_type=jnp.float32)
    acc = acc + jnp.dot(ch2, w2_ref[...], preferred_element_type=jnp.float32)
    out_ref[...] = _leaky(acc + bs_ref[...])


@functools.lru_cache(maxsize=None)
def _make_edge_pass(n, e, h):
    ept = e // (_NC * _NS)
    n_chunks = ept // _K
    zfull = 624
    mesh = plsc.VectorSubcoreMesh(core_axis_name="c", subcore_axis_name="s")

    nq = 4

    @functools.partial(
        pl.kernel,
        out_type=jax.ShapeDtypeStruct((_NC, n, h), jnp.float32),
        mesh=mesh,
        scratch_types=[
            pltpu.VMEM((nq, _K), jnp.int32),
            pltpu.VMEM((nq, _K), jnp.int32),
            pltpu.VMEM((2, _K, h), jnp.float32),
            pltpu.VMEM((2, _K, h), jnp.float32),
            pltpu.VMEM((2, _K, h // 2), jnp.int32),
            pltpu.VMEM((2, _K, h), jnp.float32),
            pltpu.VMEM_SHARED((n, h), jnp.float32),
            pltpu.SemaphoreType.DMA,
            pltpu.SemaphoreType.DMA,
            pltpu.SemaphoreType.DMA,
            pltpu.SemaphoreType.DMA,
            pltpu.SemaphoreType.DMA,
            pltpu.SemaphoreType.DMA,
            pltpu.SemaphoreType.DMA,
            pltpu.SemaphoreType.DMA,
            pltpu.SemaphoreType.DMA,
        ],
    )
    def edge_pass(a_hbm, b_hbm, c_hbm, src_hbm, dst_hbm, out_hbm,
                  idxs, idxd, bufa, bufb, bufc, bufm, acc,
                  semi, sa0, sa1, sb0, sb1, sc0, sc1, ss0, ss1):
        cid = lax.axis_index("c")
        sid = lax.axis_index("s")
        tile = cid * _NS + sid
        last = sid == _NS - 1
        row0 = sid * zfull
        ebase = tile * ept
        sem_a = (sa0, sa1)
        sem_b = (sb0, sb1)
        sem_c = (sc0, sc1)
        sem_s = (ss0, ss1)

        zero16 = jnp.zeros((16,), jnp.float32)

        def zrow(r, carry):
            for q in range(h // 16):
                bufm[0, r, pl.ds(q * 16, 16)] = zero16
            return carry

        lax.fori_loop(0, _K, zrow, 0)

        ncp = jnp.where(last, 16, 15)

        def zcp(k, carry):
            pltpu.sync_copy(bufm.at[0], acc.at[pl.ds(row0 + k * _K, _K)])
            return carry

        lax.fori_loop(0, ncp, zcp, 0)

        @pl.when(jnp.logical_not(last))
        def _zero_tail():
            pltpu.sync_copy(bufm.at[0, pl.ds(0, zfull - 15 * _K)],
                            acc.at[pl.ds(row0 + 15 * _K, zfull - 15 * _K)])

        plsc.subcore_barrier()

        def load_idx(j, q):
            di = pltpu.async_copy(
                src_hbm.at[pl.ds(ebase + j * _K, _K)], idxs.at[q], semi)
            dj = pltpu.async_copy(
                dst_hbm.at[pl.ds(ebase + j * _K, _K)], idxd.at[q], semi)
            return di, dj

        def issue_gathers(j, q, s):
            pltpu.async_copy(c_hbm.at[pl.ds(ebase + j * _K, _K)],
                             bufc.at[s], sem_c[s])
            pltpu.async_copy(a_hbm.at[idxs.at[q]], bufa.at[s], sem_a[s])
            pltpu.async_copy(b_hbm.at[idxd.at[q]], bufb.at[s], sem_b[s])

        def wait_gathers(s):
            pltpu.make_async_copy(c_hbm.at[pl.ds(0, _K)],
                                  bufc.at[s], sem_c[s]).wait()
            pltpu.make_async_copy(a_hbm.at[pl.ds(0, _K)],
                                  bufa.at[s], sem_a[s]).wait()
            pltpu.make_async_copy(b_hbm.at[pl.ds(0, _K)],
                                  bufb.at[s], sem_b[s]).wait()

        def wait_idx(q):
            pltpu.make_async_copy(src_hbm.at[pl.ds(0, _K)],
                                  idxs.at[q], semi).wait()
            pltpu.make_async_copy(src_hbm.at[pl.ds(0, _K)],
                                  idxd.at[q], semi).wait()

        def wait_scatter(s):
            pltpu.make_async_copy(c_hbm.at[pl.ds(0, _K)],
                                  bufm.at[s], sem_s[s]).wait()

        def make_chunk(j, q, s, wait_scat, pg, pi):
            sn = 1 - s
            qn = (q + 1) % nq
            if pg:
                wait_idx(qn)
                issue_gathers(j + 1, qn, sn)
            if wait_scat:
                wait_scatter(s)
            if pi:
                load_idx(j + 2, (q + 2) % nq)
            wait_gathers(s)

            himask = jnp.int32(-65536)

            @plsc.parallel_loop(0, _K, unroll=4)
            def erow(ei):
                for p in range(h // 32):
                    w = bufc[s, ei, pl.ds(16 * p, 16)]
                    clo = jax.lax.bitcast_convert_type(w << 16, jnp.float32)
                    chi = jax.lax.bitcast_convert_type(w & himask, jnp.float32)
                    sll = pl.ds(16 * p, 16)
                    slh = pl.ds(h // 2 + 16 * p, 16)
                    vl = clo + bufa[s, ei, sll] + bufb[s, ei, sll]
                    vh = chi + bufa[s, ei, slh] + bufb[s, ei, slh]
                    bufm[s, ei, sll] = jnp.maximum(vl, 0.0)
                    bufm[s, ei, slh] = jnp.maximum(vh, 0.0)

            pltpu.async_copy(bufm.at[s], acc.at[idxs.at[q]], sem_s[s],
                             add=True)

        d0, d1 = load_idx(0, 0)
        d0.wait()
        d1.wait()
        load_idx(1, 1)
        issue_gathers(0, 0, 0)
        make_chunk(0, 0, 0, False, True, True)
        make_chunk(1, 1, 1, False, True, True)

        def quad(t, carry):
            j = 2 + 4 * t
            make_chunk(j, 2, 0, True, True, True)
            make_chunk(j + 1, 3, 1, True, True, True)
            make_chunk(j + 2, 0, 0, True, True, True)
            make_chunk(j + 3, 1, 1, True, True, True)
            return carry

        lax.fori_loop(0, (n_chunks - 2) // 4 - 1, quad, 0)
        jt = n_chunks - 4
        make_chunk(jt, 2, 0, True, True, True)
        make_chunk(jt + 1, 3, 1, True, True, True)
        make_chunk(jt + 2, 0, 0, True, True, False)
        make_chunk(jt + 3, 1, 1, True, False, False)
        wait_scatter(0)
        wait_scatter(1)

        plsc.subcore_barrier()

        pltpu.sync_copy(acc.at[pl.ds(row0, zfull)],
                        out_hbm.at[cid, pl.ds(row0, zfull)])

        @pl.when(last)
        def _flush_tail():
            pltpu.sync_copy(acc.at[pl.ds(15 * zfull, n - 15 * zfull)],
                            out_hbm.at[cid, pl.ds(15 * zfull, n - 15 * zfull)])

    return edge_pass


def kernel(child_feats, edge_indices, edge_type_onehot, W1, b1, W2, b2,
           We0, be0, We1, be1, Ws, bs):
    n = child_feats.shape[1]
    e = edge_indices.shape[1]
    h = W1.shape[1]
    nfs = Ws.shape[1]

    cf = child_feats[0]
    ef = edge_type_onehot[0]
    srcdst = edge_indices[0].T
    src = srcdst[0]
    dst = srcdst[1]

    f32 = jnp.float32
    node_out = [jax.ShapeDtypeStruct((n, h), f32)] * 3

    child0, a0, b0 = pl.pallas_call(
        _node_dense_body,
        out_shape=node_out,
    )(cf, W1, b1.reshape(1, h), W2, b2.reshape(1, h),
      We0[:h], We0[h:2 * h])

    rb = 8000
    et = ef.shape[1]
    hh = h // 2
    wspec = pl.BlockSpec((et, hh), lambda i: (0, 0))
    bspec = pl.BlockSpec((1, hh), lambda i: (0, 0))
    c0, c1 = pl.pallas_call(
        _edge_dense_body,
        grid=(e // rb,),
        in_specs=[pl.BlockSpec((rb, et), lambda i: (i, 0))]
        + [wspec, wspec, bspec, bspec] * 2,
        out_specs=[pl.BlockSpec((rb, hh), lambda i: (i, 0))] * 2,
        out_shape=[jax.ShapeDtypeStruct((e, hh), jnp.int32)] * 2,
    )(ef,
      We0[2 * h:, :hh], We0[2 * h:, hh:],
      be0[:hh].reshape(1, hh), be0[hh:].reshape(1, hh),
      We1[2 * h:, :hh], We1[2 * h:, hh:],
      be1[:hh].reshape(1, hh), be1[hh:].reshape(1, hh))

    edge_pass = _make_edge_pass(n, e, h)
    p0 = edge_pass(a0, b0, c0, src, dst)

    child1, a1, b1_ = pl.pallas_call(
        _mid_dense_body,
        out_shape=node_out,
    )(p0, We1[:h], We1[h:2 * h])

    p1 = edge_pass(a1, b1_, c1, src, dst)

    out = pl.pallas_call(
        _final_dense_body,
        out_shape=jax.ShapeDtypeStruct((n, nfs), f32),
    )(child0, child1, p1, Ws[:h], Ws[h:2 * h], Ws[2 * h:], bs.reshape(1, nfs))

    return out


# speedup vs baseline: 1.0351x; 1.0001x over previous
import functools

import jax
import jax.numpy as jnp
from jax import lax
from jax.experimental import pallas as pl
from jax.experimental.pallas import tpu as pltpu
from jax.experimental.pallas import tpu_sc as plsc

_NC = 2
_NS = 16
_K = 40


def _leaky(x):
    return jnp.where(x >= 0, x, 0.1 * x)


def _node_dense_body(cf_ref, w1_ref, b1_ref, w2_ref, b2_ref, ws_ref, wd_ref,
                     child_ref, a_ref, b_ref):
    x = cf_ref[...]
    net = _leaky(_leaky(
        jnp.dot(x, w1_ref[...], preferred_element_type=jnp.float32) + b1_ref[...]))
    net = _leaky(
        jnp.dot(net, w2_ref[...], preferred_element_type=jnp.float32) + b2_ref[...])
    child_ref[...] = net
    a_ref[...] = jnp.dot(net, ws_ref[...], preferred_element_type=jnp.float32)
    b_ref[...] = jnp.dot(net, wd_ref[...], preferred_element_type=jnp.float32)


def _edge_dense_body(ef_ref, w0lo_ref, w0hi_ref, b0lo_ref, b0hi_ref,
                     w1lo_ref, w1hi_ref, b1lo_ref, b1hi_ref, c0_ref, c1_ref):
    x = ef_ref[...]

    def pack(lo, hi):
        ulo = jax.lax.bitcast_convert_type(lo, jnp.uint32)
        uhi = jax.lax.bitcast_convert_type(hi, jnp.uint32)
        ulo = (ulo + 0x8000) >> 16
        uhi = (uhi + 0x8000) >> 16
        return jax.lax.bitcast_convert_type(ulo | (uhi << 16), jnp.int32)

    def half(w_ref, b_ref):
        return jnp.dot(x, w_ref[...], preferred_element_type=jnp.float32) + b_ref[...]

    c0_ref[...] = pack(half(w0lo_ref, b0lo_ref), half(w0hi_ref, b0hi_ref))
    c1_ref[...] = pack(half(w1lo_ref, b1lo_ref), half(w1hi_ref, b1hi_ref))


def _mid_dense_body(p_ref, ws_ref, wd_ref, child_ref, a_ref, b_ref):
    ch = p_ref[0] + p_ref[1]
    child_ref[...] = ch
    a_ref[...] = jnp.dot(ch, ws_ref[...], preferred_element_type=jnp.float32)
    b_ref[...] = jnp.dot(ch, wd_ref[...], preferred_element_type=jnp.float32)


def _final_dense_body(c0_ref, c1_ref, q_ref, w0_ref, w1_ref, w2_ref, bs_ref,
                      out_ref):
    ch2 = q_ref[0] + q_ref[1]
    acc = jnp.dot(c0_ref[...], w0_ref[...], preferred_element_type=jnp.float32)
    acc = acc + jnp.dot(c1_ref[...], w1_ref[...], preferred_element_type=jnp.float32)
    acc = acc + jnp.dot(ch2, w2_ref[...], preferred_element_type=jnp.float32)
    out_ref[...] = _leaky(acc + bs_ref[...])


@functools.lru_cache(maxsize=None)
def _make_edge_pass(n, e, h):
    ept = e // (_NC * _NS)
    n_chunks = ept // _K
    zfull = 624
    mesh = plsc.VectorSubcoreMesh(core_axis_name="c", subcore_axis_name="s")

    nq = 4

    @functools.partial(
        pl.kernel,
        out_type=jax.ShapeDtypeStruct((_NC, n, h), jnp.float32),
        mesh=mesh,
        scratch_types=[
            pltpu.VMEM((nq, _K), jnp.int32),
            pltpu.VMEM((nq, _K), jnp.int32),
            pltpu.VMEM((2, _K, h), jnp.float32),
            pltpu.VMEM((2, _K, h), jnp.float32),
            pltpu.VMEM((2, _K, h // 2), jnp.int32),
            pltpu.VMEM((2, _K, h), jnp.float32),
            pltpu.VMEM_SHARED((n, h), jnp.float32),
            pltpu.SemaphoreType.DMA,
            pltpu.SemaphoreType.DMA,
            pltpu.SemaphoreType.DMA,
            pltpu.SemaphoreType.DMA,
            pltpu.SemaphoreType.DMA,
            pltpu.SemaphoreType.DMA,
            pltpu.SemaphoreType.DMA,
            pltpu.SemaphoreType.DMA,
            pltpu.SemaphoreType.DMA,
        ],
    )
    def edge_pass(a_hbm, b_hbm, c_hbm, src_hbm, dst_hbm, out_hbm,
                  idxs, idxd, bufa, bufb, bufc, bufm, acc,
                  semi, sa0, sa1, sb0, sb1, sc0, sc1, ss0, ss1):
        cid = lax.axis_index("c")
        sid = lax.axis_index("s")
        tile = cid * _NS + sid
        last = sid == _NS - 1
        row0 = sid * zfull
        ebase = tile * ept
        sem_a = (sa0, sa1)
        sem_b = (sb0, sb1)
        sem_c = (sc0, sc1)
        sem_s = (ss0, ss1)

        zero16 = jnp.zeros((16,), jnp.float32)

        def zrow(r, carry):
            for q in range(h // 16):
                bufm[0, r, pl.ds(q * 16, 16)] = zero16
            return carry

        lax.fori_loop(0, _K, zrow, 0)

        ncp = jnp.where(last, 16, 15)

        def zcp(k, carry):
            pltpu.sync_copy(bufm.at[0], acc.at[pl.ds(row0 + k * _K, _K)])
            return carry

        lax.fori_loop(0, ncp, zcp, 0)

        @pl.when(jnp.logical_not(last))
        def _zero_tail():
            pltpu.sync_copy(bufm.at[0, pl.ds(0, zfull - 15 * _K)],
                            acc.at[pl.ds(row0 + 15 * _K, zfull - 15 * _K)])

        plsc.subcore_barrier()

        def load_idx(j, q):
            di = pltpu.async_copy(
                src_hbm.at[pl.ds(ebase + j * _K, _K)], idxs.at[q], semi)
            dj = pltpu.async_copy(
                dst_hbm.at[pl.ds(ebase + j * _K, _K)], idxd.at[q], semi)
            return di, dj

        def issue_gathers(j, q, s):
            pltpu.async_copy(c_hbm.at[pl.ds(ebase + j * _K, _K)],
                             bufc.at[s], sem_c[s])
            pltpu.async_copy(a_hbm.at[idxs.at[q]], bufa.at[s], sem_a[s])
            pltpu.async_copy(b_hbm.at[idxd.at[q]], bufb.at[s], sem_b[s])

        def wait_gathers(s):
            pltpu.make_async_copy(c_hbm.at[pl.ds(0, _K)],
                                  bufc.at[s], sem_c[s]).wait()
            pltpu.make_async_copy(a_hbm.at[pl.ds(0, _K)],
                                  bufa.at[s], sem_a[s]).wait()
            pltpu.make_async_copy(b_hbm.at[pl.ds(0, _K)],
                                  bufb.at[s], sem_b[s]).wait()

        def wait_idx(q):
            pltpu.make_async_copy(src_hbm.at[pl.ds(0, _K)],
                                  idxs.at[q], semi).wait()
            pltpu.make_async_copy(src_hbm.at[pl.ds(0, _K)],
                                  idxd.at[q], semi).wait()

        def wait_scatter(s):
            pltpu.make_async_copy(c_hbm.at[pl.ds(0, _K)],
                                  bufm.at[s], sem_s[s]).wait()

        def make_chunk(j, q, s, wait_scat, pg, pi):
            sn = 1 - s
            qn = (q + 1) % nq
            if pg:
                wait_idx(qn)
                issue_gathers(j + 1, qn, sn)
            if wait_scat:
                wait_scatter(s)
            if pi:
                load_idx(j + 2, (q + 2) % nq)
            wait_gathers(s)

            himask = jnp.int32(-65536)

            @plsc.parallel_loop(0, _K, unroll=4)
            def erow(ei):
                for p in range(h // 32):
                    w = bufc[s, ei, pl.ds(16 * p, 16)]
                    clo = jax.lax.bitcast_convert_type(w << 16, jnp.float32)
                    chi = jax.lax.bitcast_convert_type(w & himask, jnp.float32)
                    sll = pl.ds(16 * p, 16)
                    slh = pl.ds(h // 2 + 16 * p, 16)
                    vl = clo + bufa[s, ei, sll] + bufb[s, ei, sll]
                    vh = chi + bufa[s, ei, slh] + bufb[s, ei, slh]
                    bufm[s, ei, sll] = jnp.maximum(vl, 0.0)
                    bufm[s, ei, slh] = jnp.maximum(vh, 0.0)

            pltpu.async_copy(bufm.at[s], acc.at[idxs.at[q]], sem_s[s],
                             add=True)

        d0, d1 = load_idx(0, 0)
        d0.wait()
        d1.wait()
        load_idx(1, 1)
        issue_gathers(0, 0, 0)
        make_chunk(0, 0, 0, False, True, True)
        make_chunk(1, 1, 1, False, True, True)

        def quad(t, carry):
            j = 2 + 4 * t
            make_chunk(j, 2, 0, True, True, True)
            make_chunk(j + 1, 3, 1, True, True, True)
            make_chunk(j + 2, 0, 0, True, True, True)
            make_chunk(j + 3, 1, 1, True, True, True)
            return carry

        lax.fori_loop(0, (n_chunks - 2) // 4 - 1, quad, 0)
        jt = n_chunks - 4
        make_chunk(jt, 2, 0, True, True, True)
        make_chunk(jt + 1, 3, 1, True, True, True)
        make_chunk(jt + 2, 0, 0, True, True, False)
        make_chunk(jt + 3, 1, 1, True, False, False)
        wait_scatter(0)
        wait_scatter(1)

        plsc.subcore_barrier()

        pltpu.sync_copy(acc.at[pl.ds(row0, zfull)],
                        out_hbm.at[cid, pl.ds(row0, zfull)])

        @pl.when(last)
        def _flush_tail():
            pltpu.sync_copy(acc.at[pl.ds(15 * zfull, n - 15 * zfull)],
                            out_hbm.at[cid, pl.ds(15 * zfull, n - 15 * zfull)])

    return edge_pass


def kernel(child_feats, edge_indices, edge_type_onehot, W1, b1, W2, b2,
           We0, be0, We1, be1, Ws, bs):
    n = child_feats.shape[1]
    e = edge_indices.shape[1]
    h = W1.shape[1]
    nfs = Ws.shape[1]

    cf = child_feats[0]
    ef = edge_type_onehot[0]
    srcdst = edge_indices[0].T
    src = srcdst[0]
    dst = srcdst[1]

    f32 = jnp.float32
    node_out = [jax.ShapeDtypeStruct((n, h), f32)] * 3

    child0, a0, b0 = pl.pallas_call(
        _node_dense_body,
        out_shape=node_out,
    )(cf, W1, b1.reshape(1, h), W2, b2.reshape(1, h),
      We0[:h], We0[h:2 * h])

    rb = 8000
    et = ef.shape[1]
    hh = h // 2
    wspec = pl.BlockSpec((et, hh), lambda i: (0, 0))
    bspec = pl.BlockSpec((1, hh), lambda i: (0, 0))
    c0, c1 = pl.pallas_call(
        _edge_dense_body,
        grid=(e // rb,),
        in_specs=[pl.BlockSpec((rb, et), lambda i: (i, 0))]
        + [wspec, wspec, bspec, bspec] * 2,
        out_specs=[pl.BlockSpec((rb, hh), lambda i: (i, 0))] * 2,
        out_shape=[jax.ShapeDtypeStruct((e, hh), jnp.int32)] * 2,
    )(ef,
      We0[2 * h:, :hh], We0[2 * h:, hh:],
      be0[:hh].reshape(1, hh), be0[hh:].reshape(1, hh),
      We1[2 * h:, :hh], We1[2 * h:, hh:],
      be1[:hh].reshape(1, hh), be1[hh:].reshape(1, hh))

    edge_pass = _make_edge_pass(n, e, h)
    p0 = edge_pass(a0, b0, c0, src, dst)

    child1, a1, b1_ = pl.pallas_call(
        _mid_dense_body,
        out_shape=node_out,
    )(p0, We1[:h], We1[h:2 * h])

    p1 = edge_pass(a1, b1_, c1, src, dst)

    out = pl.pallas_call(
        _final_dense_body,
        out_shape=jax.ShapeDtypeStruct((n, nfs), f32),
    )(child0, child1, p1, Ws[:h], Ws[h:2 * h], Ws[2 * h:], bs.reshape(1, nfs))

    return out
